# Initial kernel scaffold; baseline (speedup 1.0000x reference)
import jax
import jax.numpy as jnp
from jax.experimental import pallas as pl


def kernel(z0, z1, proj_fused_w, proj_fused_b, upT_0, wT_0, b_col_0, upT_1, wT_1, b_col_1, upT_2, wT_2, b_col_2):
    raise NotImplementedError("write your pallas kernel here")


# speedup vs baseline: 2.0792x; 2.0792x over previous
import functools

import jax
import jax.numpy as jnp
from jax import lax
from jax.experimental import pallas as pl
from jax.experimental.pallas import tpu as pltpu


def _elu(x):
    return jnp.where(x > 0.0, x, jnp.exp(jnp.minimum(x, 0.0)) - 1.0)


def _layer(X, w_ref, m_ref, up_ref, b_ref, y_scr, B, elu):
    S, C_out, C_in = w_ref.shape
    V_in = X.shape[1]
    V_out = up_ref.shape[2]
    mask = m_ref[...]
    for s in range(S):
        w = w_ref[s]
        wrow = jnp.concatenate([w] * B, axis=1)
        wt = jnp.concatenate([wrow] * B, axis=0)
        Wb = wt * mask
        y_scr[:, s * V_in:(s + 1) * V_in] = jnp.dot(
            Wb, X, preferred_element_type=jnp.float32)
    Up = up_ref[...].reshape(S * V_in, V_out)
    O = jnp.dot(y_scr[...], Up, preferred_element_type=jnp.float32) + b_ref[...]
    return _elu(O) if elu else O


def _decoder_kernel(zexp_ref, wp_ref, bp_ref,
                    w0_ref, m0_ref, up0_ref, b0_ref,
                    w1_ref, m1_ref, up1_ref, b1_ref,
                    w2_ref, m2_ref, up2_ref, b2_ref,
                    o_ref, y0_scr, y1_scr, y2_scr, *, B):
    X = jnp.dot(zexp_ref[...], wp_ref[...],
                preferred_element_type=jnp.float32) + bp_ref[...]
    X = _layer(X, w0_ref, m0_ref, up0_ref, b0_ref, y0_scr, B, elu=True)
    X = _layer(X, w1_ref, m1_ref, up1_ref, b1_ref, y1_scr, B, elu=True)
    X = _layer(X, w2_ref, m2_ref, up2_ref, b2_ref, y2_scr, B, elu=False)
    C_out = o_ref.shape[1]
    o_ref[...] = X.reshape(B, C_out, X.shape[1])


def kernel(z0, z1, proj_fused_w, proj_fused_b,
           upT_0, wT_0, b_col_0,
           upT_1, wT_1, b_col_1,
           upT_2, wT_2, b_col_2):
    B = z0.shape[0]
    S, V0, V1 = upT_0.shape
    V2 = upT_1.shape[2]
    V3 = upT_2.shape[2]
    Z = proj_fused_w.shape[0]
    C0 = proj_fused_w.shape[1] // V0
    C1, C2, C3 = wT_0.shape[1], wT_1.shape[1], wT_2.shape[1]
    f32 = jnp.float32

    z = jnp.concatenate([z0, z1], axis=-1)
    zexp = jnp.kron(z, jnp.eye(C0, dtype=f32))
    wp = proj_fused_w.reshape(Z * C0, V0)
    bp = jnp.tile(proj_fused_b.reshape(C0, V0), (B, 1))

    def blk_mask(co, ci):
        r = jnp.arange(B * co)[:, None] // co
        c = jnp.arange(B * ci)[None, :] // ci
        return (r == c).astype(f32)

    m0, m1, m2 = blk_mask(C1, C0), blk_mask(C2, C1), blk_mask(C3, C2)
    bt0 = jnp.tile(b_col_0, (B, 1))
    bt1 = jnp.tile(b_col_1, (B, 1))
    bt2 = jnp.tile(b_col_2, (B, 1))

    NC = 2
    Vc = V3 // NC

    out = pl.pallas_call(
        functools.partial(_decoder_kernel, B=B),
        out_shape=jax.ShapeDtypeStruct((B, C3, V3), f32),
        grid=(NC,),
        in_specs=[
            pl.BlockSpec((B * C0, Z * C0), lambda i: (0, 0)),
            pl.BlockSpec((Z * C0, V0), lambda i: (0, 0)),
            pl.BlockSpec((B * C0, V0), lambda i: (0, 0)),
            pl.BlockSpec((S, C1, C0), lambda i: (0, 0, 0)),
            pl.BlockSpec((B * C1, B * C0), lambda i: (0, 0)),
            pl.BlockSpec((S, V0, V1), lambda i: (0, 0, 0)),
            pl.BlockSpec((B * C1, 1), lambda i: (0, 0)),
            pl.BlockSpec((S, C2, C1), lambda i: (0, 0, 0)),
            pl.BlockSpec((B * C2, B * C1), lambda i: (0, 0)),
            pl.BlockSpec((S, V1, V2), lambda i: (0, 0, 0)),
            pl.BlockSpec((B * C2, 1), lambda i: (0, 0)),
            pl.BlockSpec((S, C3, C2), lambda i: (0, 0, 0)),
            pl.BlockSpec((B * C3, B * C2), lambda i: (0, 0)),
            pl.BlockSpec((S, V2, Vc), lambda i: (0, 0, i)),
            pl.BlockSpec((B * C3, 1), lambda i: (0, 0)),
        ],
        out_specs=pl.BlockSpec((B, C3, Vc), lambda i: (0, 0, i)),
        scratch_shapes=[
            pltpu.VMEM((B * C1, S * V0), f32),
            pltpu.VMEM((B * C2, S * V1), f32),
            pltpu.VMEM((B * C3, S * V2), f32),
        ],
        compiler_params=pltpu.CompilerParams(
            dimension_semantics=("parallel",),
        ),
    )(zexp, wp, bp,
      wT_0, m0, upT_0, bt0,
      wT_1, m1, upT_1, bt1,
      wT_2, m2, upT_2, bt2)

    return jnp.transpose(out, (0, 2, 1))


# speedup vs baseline: 2.4642x; 1.1852x over previous
import functools

import numpy as np
import jax
import jax.numpy as jnp
from jax.experimental import pallas as pl
from jax.experimental.pallas import tpu as pltpu


def _elu(x):
    return jnp.where(x > 0.0, x, jnp.exp(jnp.minimum(x, 0.0)) - 1.0)


def _layer(X, w_ref, m_ref, up_ref, b_ref, y_scr, B, elu):
    S, C_out, C_in = w_ref.shape
    V_in = X.shape[1]
    V_out = up_ref.shape[2]
    mask = m_ref[...]
    for s in range(S):
        w = w_ref[s]
        wrow = jnp.concatenate([w] * B, axis=1)
        wt = jnp.concatenate([wrow] * B, axis=0)
        Wb = wt * mask
        y_scr[:, s * V_in:(s + 1) * V_in] = jnp.dot(
            Wb, X, preferred_element_type=jnp.float32)
    Up = up_ref[...].reshape(S * V_in, V_out)
    bias = jnp.concatenate([b_ref[...]] * B, axis=0)
    O = jnp.dot(y_scr[...], Up, preferred_element_type=jnp.float32) + bias
    return _elu(O) if elu else O


def _decoder_kernel(z0_ref, z1_ref, wp_ref, bp_ref,
                    w0_ref, m0_ref, up0_ref, b0_ref,
                    w1_ref, m1_ref, up1_ref, b1_ref,
                    w2_ref, m2_ref, up2_ref, b2_ref,
                    o_ref, y0_scr, y1_scr, y2_scr, *, B, C0, V0):
    z = jnp.concatenate([z0_ref[...], z1_ref[...]], axis=1)
    y = jnp.dot(z, wp_ref[...], preferred_element_type=jnp.float32) + bp_ref[...]
    X = y.reshape(B * C0, V0)
    X = _layer(X, w0_ref, m0_ref, up0_ref, b0_ref, y0_scr, B, elu=True)
    X = _layer(X, w1_ref, m1_ref, up1_ref, b1_ref, y1_scr, B, elu=True)
    X = _layer(X, w2_ref, m2_ref, up2_ref, b2_ref, y2_scr, B, elu=False)
    C_out = o_ref.shape[2]
    for b in range(B):
        o_ref[b] = X[b * C_out:(b + 1) * C_out, :].T


def kernel(z0, z1, proj_fused_w, proj_fused_b,
           upT_0, wT_0, b_col_0,
           upT_1, wT_1, b_col_1,
           upT_2, wT_2, b_col_2):
    B = z0.shape[0]
    S, V0, V1 = upT_0.shape
    V2 = upT_1.shape[2]
    V3 = upT_2.shape[2]
    Z = proj_fused_w.shape[0]
    C0 = proj_fused_w.shape[1] // V0
    C1, C2, C3 = wT_0.shape[1], wT_1.shape[1], wT_2.shape[1]
    f32 = jnp.float32

    def blk_mask(co, ci):
        r = np.arange(B * co)[:, None] // co
        c = np.arange(B * ci)[None, :] // ci
        return jnp.asarray((r == c).astype(np.float32))

    m0, m1, m2 = blk_mask(C1, C0), blk_mask(C2, C1), blk_mask(C3, C2)

    NC = 2
    Vc = V3 // NC

    return pl.pallas_call(
        functools.partial(_decoder_kernel, B=B, C0=C0, V0=V0),
        out_shape=jax.ShapeDtypeStruct((B, V3, C3), f32),
        grid=(NC,),
        in_specs=[
            pl.BlockSpec((B, z0.shape[1]), lambda i: (0, 0)),
            pl.BlockSpec((B, z1.shape[1]), lambda i: (0, 0)),
            pl.BlockSpec((Z, C0 * V0), lambda i: (0, 0)),
            pl.BlockSpec((1, C0 * V0), lambda i: (0, 0)),
            pl.BlockSpec((S, C1, C0), lambda i: (0, 0, 0)),
            pl.BlockSpec((B * C1, B * C0), lambda i: (0, 0)),
            pl.BlockSpec((S, V0, V1), lambda i: (0, 0, 0)),
            pl.BlockSpec((C1, 1), lambda i: (0, 0)),
            pl.BlockSpec((S, C2, C1), lambda i: (0, 0, 0)),
            pl.BlockSpec((B * C2, B * C1), lambda i: (0, 0)),
            pl.BlockSpec((S, V1, V2), lambda i: (0, 0, 0)),
            pl.BlockSpec((C2, 1), lambda i: (0, 0)),
            pl.BlockSpec((S, C3, C2), lambda i: (0, 0, 0)),
            pl.BlockSpec((B * C3, B * C2), lambda i: (0, 0)),
            pl.BlockSpec((S, V2, Vc), lambda i: (0, 0, i)),
            pl.BlockSpec((C3, 1), lambda i: (0, 0)),
        ],
        out_specs=pl.BlockSpec((B, Vc, C3), lambda i: (0, i, 0)),
        scratch_shapes=[
            pltpu.VMEM((B * C1, S * V0), f32),
            pltpu.VMEM((B * C2, S * V1), f32),
            pltpu.VMEM((B * C3, S * V2), f32),
        ],
        compiler_params=pltpu.CompilerParams(
            dimension_semantics=("parallel",),
        ),
    )(z0, z1, proj_fused_w, proj_fused_b,
      wT_0, m0, upT_0, b_col_0,
      wT_1, m1, upT_1, b_col_1,
      wT_2, m2, upT_2, b_col_2)
